# baseline (speedup 1.0000x reference)
import functools

import jax
import jax.numpy as jnp
from jax import lax
from jax.experimental import pallas as pl
from jax.experimental.pallas import tpu as pltpu
from jax.experimental.pallas import tpu_sc as plsc

NU = 5000
NI = 5000
NE = 320000
D = 128
NLAYERS = 3

NC = 2
NS = 16
NW = NC * NS
FPT = D // NW
NPAD = 5120

CH_BC = 3200
NCH_BC = NE // CH_BC
CH_NRM = 2000
EPT = NE // NW
CH_L = 3200
NCH_L = NE // CH_L
GPT = 4096 // NW
UNROLL = 4

_mesh = functools.partial(
    plsc.VectorSubcoreMesh, core_axis_name="c", subcore_axis_name="s")
_sc_params = pltpu.CompilerParams(needs_layout_passes=False)


def _wid():
    return lax.axis_index("s") * NC + lax.axis_index("c")


def _rsqrt16(x):
    iv = plsc.bitcast(x, jnp.int32)
    iv = jnp.int32(0x5F3759DF) - (iv >> 1)
    y = plsc.bitcast(iv, jnp.float32)
    for _ in range(3):
        y = y * (1.5 - 0.5 * x * y * y)
    return y


@functools.partial(
    pl.kernel,
    out_type=[jax.ShapeDtypeStruct((NW, NPAD), jnp.float32),
              jax.ShapeDtypeStruct((NW, NPAD), jnp.float32)],
    mesh=_mesh(),
    compiler_params=_sc_params,
    scratch_types=[
        pltpu.VMEM((NPAD,), jnp.float32),
        pltpu.VMEM((NPAD,), jnp.float32),
        pltpu.VMEM((EPT,), jnp.int32),
        pltpu.VMEM((EPT,), jnp.int32),
        pltpu.SemaphoreType.DMA,
    ],
)
def _bincount(src_hbm, dst_hbm, pu_hbm, pi_hbm, degu_v, degi_v, src_b,
              dst_b, sem):
    wid = _wid()
    base = wid * EPT
    pltpu.async_copy(src_hbm.at[pl.ds(base, EPT)], src_b, sem)
    pltpu.async_copy(dst_hbm.at[pl.ds(base, EPT)], dst_b, sem)

    zeros16 = jnp.zeros((16,), jnp.float32)
    ones16 = jnp.ones((16,), jnp.float32)

    @plsc.parallel_loop(0, NPAD, step=16, unroll=4)
    def _zero(j):
        degu_v[pl.ds(j, 16)] = zeros16
        degi_v[pl.ds(j, 16)] = zeros16

    pltpu.make_async_copy(src_hbm.at[pl.ds(0, EPT)], src_b, sem).wait()
    pltpu.make_async_copy(src_hbm.at[pl.ds(0, EPT)], dst_b, sem).wait()

    @plsc.parallel_loop(0, EPT, step=16, unroll=4)
    def _grp(j):
        s = src_b[pl.ds(j, 16)]
        d = dst_b[pl.ds(j, 16)]
        plsc.addupdate_scatter(degu_v, [s], ones16)
        plsc.addupdate_scatter(degi_v, [d], ones16)

    pltpu.sync_copy(degu_v, pu_hbm.at[wid])
    pltpu.sync_copy(degi_v, pi_hbm.at[wid])


SEG = NPAD // 8


@functools.partial(
    pl.kernel,
    out_type=[jax.ShapeDtypeStruct((NPAD,), jnp.float32),
              jax.ShapeDtypeStruct((NPAD,), jnp.float32)],
    mesh=_mesh(),
    compiler_params=_sc_params,
    scratch_types=[
        pltpu.VMEM((NW, SEG), jnp.float32),
        pltpu.VMEM((SEG,), jnp.float32),
    ],
)
def _degreduce(pu_hbm, pi_hbm, du_hbm, di_hbm, buf, acc):
    wid = _wid()

    def reduce_one(part_hbm, out_hbm, seg0):
        pltpu.sync_copy(part_hbm.at[:, pl.ds(seg0, SEG)], buf)

        @plsc.parallel_loop(0, SEG, step=16, unroll=2)
        def _red(j):
            t = buf[0, pl.ds(j, 16)]
            for p in range(1, NW):
                t = t + buf[p, pl.ds(j, 16)]
            acc[pl.ds(j, 16)] = t

        pltpu.sync_copy(acc, out_hbm.at[pl.ds(seg0, SEG)])

    @pl.when(wid < 8)
    def _():
        reduce_one(pu_hbm, du_hbm, wid * SEG)

    @pl.when(jnp.logical_and(wid >= 8, wid < 16))
    def _():
        reduce_one(pi_hbm, di_hbm, (wid - 8) * SEG)


@functools.partial(
    pl.kernel,
    out_type=jax.ShapeDtypeStruct((NE,), jnp.float32),
    mesh=_mesh(),
    compiler_params=_sc_params,
    scratch_types=[
        pltpu.VMEM((NPAD,), jnp.float32),
        pltpu.VMEM((NPAD,), jnp.float32),
        pltpu.VMEM((EPT,), jnp.int32),
        pltpu.VMEM((EPT,), jnp.int32),
        pltpu.VMEM((EPT,), jnp.float32),
        pltpu.SemaphoreType.DMA,
    ],
)
def _edgenorm(src_hbm, dst_hbm, du_hbm, di_hbm, nrm_hbm, degu_v, degi_v,
              src_b, dst_b, out_b, sem):
    wid = _wid()
    base = wid * EPT
    pltpu.async_copy(src_hbm.at[pl.ds(base, EPT)], src_b, sem)
    pltpu.async_copy(dst_hbm.at[pl.ds(base, EPT)], dst_b, sem)
    pltpu.async_copy(du_hbm, degu_v, sem)
    pltpu.async_copy(di_hbm, degi_v, sem)
    pltpu.make_async_copy(src_hbm.at[pl.ds(0, EPT)], src_b, sem).wait()
    pltpu.make_async_copy(src_hbm.at[pl.ds(0, EPT)], dst_b, sem).wait()
    pltpu.make_async_copy(du_hbm, degu_v, sem).wait()
    pltpu.make_async_copy(di_hbm, degi_v, sem).wait()

    @plsc.parallel_loop(0, EPT, step=16, unroll=4)
    def _grp(j):
        s = src_b[pl.ds(j, 16)]
        d = dst_b[pl.ds(j, 16)]
        du = plsc.load_gather(degu_v, [s])
        di = plsc.load_gather(degi_v, [d])
        out_b[pl.ds(j, 16)] = _rsqrt16(du * di)

    pltpu.sync_copy(out_b, nrm_hbm.at[pl.ds(base, EPT)])


@functools.partial(
    pl.kernel,
    out_type=[jax.ShapeDtypeStruct((D, NPAD), jnp.float32),
              jax.ShapeDtypeStruct((D, NPAD), jnp.float32)],
    mesh=_mesh(),
    compiler_params=_sc_params,
    scratch_types=[
        pltpu.VMEM((FPT, NPAD), jnp.float32),
        pltpu.VMEM((FPT, NPAD), jnp.float32),
        pltpu.VMEM((FPT, NPAD), jnp.float32),
        pltpu.VMEM((FPT, NPAD), jnp.float32),
        pltpu.VMEM((2, 2, CH_L), jnp.int32),
        pltpu.VMEM((2, CH_L), jnp.float32),
        pltpu.SemaphoreType.DMA,
        pltpu.SemaphoreType.DMA,
    ],
)
def _layer(hu_hbm, hi_hbm, src_hbm, dst_hbm, nrm_hbm, aggu_hbm, aggi_hbm,
           hu_v, hi_v, aggu_v, aggi_v, ebuf, nbuf, sem0, sem1):
    wid = _wid()
    r0 = wid * FPT
    sems = (sem0, sem1)

    def issue(ci, b):
        off = ci * CH_L
        pltpu.async_copy(src_hbm.at[pl.ds(off, CH_L)], ebuf.at[b, 0],
                         sems[b])
        pltpu.async_copy(dst_hbm.at[pl.ds(off, CH_L)], ebuf.at[b, 1],
                         sems[b])
        pltpu.async_copy(nrm_hbm.at[pl.ds(off, CH_L)], nbuf.at[b],
                         sems[b])

    issue(0, 0)
    pltpu.sync_copy(hu_hbm.at[pl.ds(r0, FPT)], hu_v)
    pltpu.sync_copy(hi_hbm.at[pl.ds(r0, FPT)], hi_v)

    zeros16 = jnp.zeros((16,), jnp.float32)

    @plsc.parallel_loop(0, NPAD, step=16, unroll=4)
    def _zero(j):
        for f in range(FPT):
            aggu_v[f, pl.ds(j, 16)] = zeros16
            aggi_v[f, pl.ds(j, 16)] = zeros16

    cvs = [jnp.full((16,), f, jnp.int32) for f in range(FPT)]

    def do_chunk(ci, b):
        @pl.when(ci + 1 < NCH_L)
        def _():
            issue(ci + 1, 1 - b)
        pltpu.make_async_copy(src_hbm.at[pl.ds(0, CH_L)], ebuf.at[b, 0],
                              sems[b]).wait()
        pltpu.make_async_copy(src_hbm.at[pl.ds(0, CH_L)], ebuf.at[b, 1],
                              sems[b]).wait()
        pltpu.make_async_copy(nrm_hbm.at[pl.ds(0, CH_L)], nbuf.at[b],
                              sems[b]).wait()

        @plsc.parallel_loop(0, CH_L, step=16, unroll=UNROLL)
        def _grp(j):
            s = ebuf[b, 0, pl.ds(j, 16)]
            d = ebuf[b, 1, pl.ds(j, 16)]
            n = nbuf[b, pl.ds(j, 16)]
            hu = [plsc.load_gather(hu_v, [cvs[f], s]) for f in range(FPT)]
            hi = [plsc.load_gather(hi_v, [cvs[f], d]) for f in range(FPT)]
            for f in range(FPT):
                plsc.addupdate_scatter(aggi_v, [cvs[f], d], n * hu[f])
                plsc.addupdate_scatter(aggu_v, [cvs[f], s], n * hi[f])

    def pair(p, c):
        do_chunk(2 * p, 0)
        do_chunk(2 * p + 1, 1)
        return c
    lax.fori_loop(0, NCH_L // 2, pair, 0)

    pltpu.sync_copy(aggu_v, aggu_hbm.at[pl.ds(r0, FPT)])
    pltpu.sync_copy(aggi_v, aggi_hbm.at[pl.ds(r0, FPT)])


def _norm_body(scale, aggu_ref, aggi_ref, embu_ref, embi_ref,
               hu_out, hi_out, embu_out, embi_out):
    for agg_ref, emb_ref, h_out, e_out in (
            (aggu_ref, embu_ref, hu_out, embu_out),
            (aggi_ref, embi_ref, hi_out, embi_out)):
        x = agg_ref[...]
        nrm = jnp.sqrt(jnp.sum(x * x, axis=0, keepdims=True))
        h = x / jnp.maximum(nrm, 1e-12)
        h_out[...] = h
        e_out[...] = emb_ref[...] + h * scale


def _norm_call(scale):
    shp = jax.ShapeDtypeStruct((D, NPAD), jnp.float32)
    return pl.pallas_call(
        functools.partial(_norm_body, scale),
        out_shape=[shp, shp, shp, shp],
    )


@functools.partial(
    pl.kernel,
    out_type=[jax.ShapeDtypeStruct((4096, D), jnp.float32)] * 3,
    mesh=_mesh(),
    compiler_params=_sc_params,
    scratch_types=[
        pltpu.VMEM((GPT,), jnp.int32),
        pltpu.VMEM((GPT, D), jnp.float32),
        pltpu.SemaphoreType.DMA,
    ],
)
def _lookup(embu_hbm, embi_hbm, users_hbm, pos_hbm, neg_hbm,
            ug_hbm, pg_hbm, ng_hbm, idx_v, rows_v, sem):
    wid = _wid()
    base = wid * GPT
    for tab, idx_hbm, out_hbm in ((embu_hbm, users_hbm, ug_hbm),
                                  (embi_hbm, pos_hbm, pg_hbm),
                                  (embi_hbm, neg_hbm, ng_hbm)):
        pltpu.sync_copy(idx_hbm.at[pl.ds(base, GPT)], idx_v)
        pltpu.async_copy(tab.at[idx_v], rows_v, sem).wait()
        pltpu.sync_copy(rows_v, out_hbm.at[pl.ds(base, GPT)])


def kernel(user_feat, item_feat, edge_src, edge_dst, users, pos_items,
           neg_items):
    pad = ((0, 0), (0, NPAD - NU))
    hu_t = jnp.pad(user_feat.T, pad)
    hi_t = jnp.pad(item_feat.T, pad)

    pu, pi = _bincount(edge_src, edge_dst)
    du, di = _degreduce(pu, pi)
    norm_e = _edgenorm(edge_src, edge_dst, du, di)

    embu_t, embi_t = hu_t, hi_t
    for k in range(NLAYERS):
        aggu_t, aggi_t = _layer(hu_t, hi_t, edge_src, edge_dst, norm_e)
        hu_t, hi_t, embu_t, embi_t = _norm_call(1.0 / (k + 1))(
            aggu_t, aggi_t, embu_t, embi_t)

    embu = embu_t[:, :NU].T
    embi = embi_t[:, :NI].T
    return tuple(_lookup(embu, embi, users, pos_items, neg_items))


# speedup vs baseline: 1.0737x; 1.0737x over previous
import functools

import jax
import jax.numpy as jnp
from jax import lax
from jax.experimental import pallas as pl
from jax.experimental.pallas import tpu as pltpu
from jax.experimental.pallas import tpu_sc as plsc

NU = 5000
NI = 5000
NE = 320000
D = 128
NLAYERS = 3

NC = 2
NS = 16
NW = NC * NS
FPT = D // NW
NPAD = 5120

CH_BC = 3200
NCH_BC = NE // CH_BC
CH_NRM = 2000
EPT = NE // NW
CH_L = 3200
NCH_L = NE // CH_L
GPT = 4096 // NW
UNROLL = 4

_mesh = functools.partial(
    plsc.VectorSubcoreMesh, core_axis_name="c", subcore_axis_name="s")
_sc_params = pltpu.CompilerParams(needs_layout_passes=False)


def _wid():
    return lax.axis_index("s") * NC + lax.axis_index("c")


def _rsqrt16(x):
    iv = plsc.bitcast(x, jnp.int32)
    iv = jnp.int32(0x5F3759DF) - (iv >> 1)
    y = plsc.bitcast(iv, jnp.float32)
    for _ in range(3):
        y = y * (1.5 - 0.5 * x * y * y)
    return y


@functools.partial(
    pl.kernel,
    out_type=[jax.ShapeDtypeStruct((NW, NPAD), jnp.float32),
              jax.ShapeDtypeStruct((NW, NPAD), jnp.float32)],
    mesh=_mesh(),
    compiler_params=_sc_params,
    scratch_types=[
        pltpu.VMEM((NPAD,), jnp.float32),
        pltpu.VMEM((NPAD,), jnp.float32),
        pltpu.VMEM((EPT,), jnp.int32),
        pltpu.VMEM((EPT,), jnp.int32),
        pltpu.SemaphoreType.DMA,
    ],
)
def _bincount(src_hbm, dst_hbm, pu_hbm, pi_hbm, degu_v, degi_v, src_b,
              dst_b, sem):
    wid = _wid()
    base = wid * EPT
    pltpu.async_copy(src_hbm.at[pl.ds(base, EPT)], src_b, sem)
    pltpu.async_copy(dst_hbm.at[pl.ds(base, EPT)], dst_b, sem)

    zeros16 = jnp.zeros((16,), jnp.float32)
    ones16 = jnp.ones((16,), jnp.float32)

    @plsc.parallel_loop(0, NPAD, step=16, unroll=4)
    def _zero(j):
        degu_v[pl.ds(j, 16)] = zeros16
        degi_v[pl.ds(j, 16)] = zeros16

    pltpu.make_async_copy(src_hbm.at[pl.ds(0, EPT)], src_b, sem).wait()
    pltpu.make_async_copy(src_hbm.at[pl.ds(0, EPT)], dst_b, sem).wait()

    @plsc.parallel_loop(0, EPT, step=16, unroll=4)
    def _grp(j):
        s = src_b[pl.ds(j, 16)]
        d = dst_b[pl.ds(j, 16)]
        plsc.addupdate_scatter(degu_v, [s], ones16)
        plsc.addupdate_scatter(degi_v, [d], ones16)

    pltpu.sync_copy(degu_v, pu_hbm.at[wid])
    pltpu.sync_copy(degi_v, pi_hbm.at[wid])


SEG = NPAD // 8


@functools.partial(
    pl.kernel,
    out_type=[jax.ShapeDtypeStruct((NPAD,), jnp.float32),
              jax.ShapeDtypeStruct((NPAD,), jnp.float32)],
    mesh=_mesh(),
    compiler_params=_sc_params,
    scratch_types=[
        pltpu.VMEM((NW, SEG), jnp.float32),
        pltpu.VMEM((SEG,), jnp.float32),
    ],
)
def _degreduce(pu_hbm, pi_hbm, du_hbm, di_hbm, buf, acc):
    wid = _wid()

    def reduce_one(part_hbm, out_hbm, seg0):
        pltpu.sync_copy(part_hbm.at[:, pl.ds(seg0, SEG)], buf)

        @plsc.parallel_loop(0, SEG, step=16, unroll=2)
        def _red(j):
            t = buf[0, pl.ds(j, 16)]
            for p in range(1, NW):
                t = t + buf[p, pl.ds(j, 16)]
            acc[pl.ds(j, 16)] = t

        pltpu.sync_copy(acc, out_hbm.at[pl.ds(seg0, SEG)])

    @pl.when(wid < 8)
    def _():
        reduce_one(pu_hbm, du_hbm, wid * SEG)

    @pl.when(jnp.logical_and(wid >= 8, wid < 16))
    def _():
        reduce_one(pi_hbm, di_hbm, (wid - 8) * SEG)


@functools.partial(
    pl.kernel,
    out_type=jax.ShapeDtypeStruct((NE,), jnp.float32),
    mesh=_mesh(),
    compiler_params=_sc_params,
    scratch_types=[
        pltpu.VMEM((NPAD,), jnp.float32),
        pltpu.VMEM((NPAD,), jnp.float32),
        pltpu.VMEM((EPT,), jnp.int32),
        pltpu.VMEM((EPT,), jnp.int32),
        pltpu.VMEM((EPT,), jnp.float32),
        pltpu.SemaphoreType.DMA,
    ],
)
def _edgenorm(src_hbm, dst_hbm, du_hbm, di_hbm, nrm_hbm, degu_v, degi_v,
              src_b, dst_b, out_b, sem):
    wid = _wid()
    base = wid * EPT
    pltpu.async_copy(src_hbm.at[pl.ds(base, EPT)], src_b, sem)
    pltpu.async_copy(dst_hbm.at[pl.ds(base, EPT)], dst_b, sem)
    pltpu.async_copy(du_hbm, degu_v, sem)
    pltpu.async_copy(di_hbm, degi_v, sem)
    pltpu.make_async_copy(src_hbm.at[pl.ds(0, EPT)], src_b, sem).wait()
    pltpu.make_async_copy(src_hbm.at[pl.ds(0, EPT)], dst_b, sem).wait()
    pltpu.make_async_copy(du_hbm, degu_v, sem).wait()
    pltpu.make_async_copy(di_hbm, degi_v, sem).wait()

    @plsc.parallel_loop(0, EPT, step=16, unroll=4)
    def _grp(j):
        s = src_b[pl.ds(j, 16)]
        d = dst_b[pl.ds(j, 16)]
        du = plsc.load_gather(degu_v, [s])
        di = plsc.load_gather(degi_v, [d])
        out_b[pl.ds(j, 16)] = _rsqrt16(du * di)

    pltpu.sync_copy(out_b, nrm_hbm.at[pl.ds(base, EPT)])


@functools.partial(
    pl.kernel,
    out_type=[jax.ShapeDtypeStruct((D, NPAD), jnp.float32),
              jax.ShapeDtypeStruct((D, NPAD), jnp.float32)],
    mesh=_mesh(),
    compiler_params=_sc_params,
    scratch_types=[
        pltpu.VMEM((FPT, NPAD), jnp.float32),
        pltpu.VMEM((FPT, NPAD), jnp.float32),
        pltpu.VMEM((FPT, NPAD), jnp.float32),
        pltpu.VMEM((FPT, NPAD), jnp.float32),
        pltpu.VMEM((2, 2, CH_L), jnp.int32),
        pltpu.VMEM((2, CH_L), jnp.float32),
        pltpu.SemaphoreType.DMA,
        pltpu.SemaphoreType.DMA,
    ],
)
def _layer(hu_hbm, hi_hbm, src_hbm, dst_hbm, nrm_hbm, aggu_hbm, aggi_hbm,
           hu_v, hi_v, aggu_v, aggi_v, ebuf, nbuf, sem0, sem1):
    wid = _wid()
    r0 = wid * FPT
    sems = (sem0, sem1)

    def issue(ci, b):
        off = ci * CH_L
        pltpu.async_copy(src_hbm.at[pl.ds(off, CH_L)], ebuf.at[b, 0],
                         sems[b])
        pltpu.async_copy(dst_hbm.at[pl.ds(off, CH_L)], ebuf.at[b, 1],
                         sems[b])
        pltpu.async_copy(nrm_hbm.at[pl.ds(off, CH_L)], nbuf.at[b],
                         sems[b])

    issue(0, 0)
    pltpu.sync_copy(hu_hbm.at[pl.ds(r0, FPT)], hu_v)
    pltpu.sync_copy(hi_hbm.at[pl.ds(r0, FPT)], hi_v)

    zeros16 = jnp.zeros((16,), jnp.float32)

    @plsc.parallel_loop(0, NPAD, step=16, unroll=4)
    def _zero(j):
        for f in range(FPT):
            aggu_v[f, pl.ds(j, 16)] = zeros16
            aggi_v[f, pl.ds(j, 16)] = zeros16

    cvs = [jnp.full((16,), f, jnp.int32) for f in range(FPT)]

    def do_chunk(ci, b):
        @pl.when(ci + 1 < NCH_L)
        def _():
            issue(ci + 1, 1 - b)
        pltpu.make_async_copy(src_hbm.at[pl.ds(0, CH_L)], ebuf.at[b, 0],
                              sems[b]).wait()
        pltpu.make_async_copy(src_hbm.at[pl.ds(0, CH_L)], ebuf.at[b, 1],
                              sems[b]).wait()
        pltpu.make_async_copy(nrm_hbm.at[pl.ds(0, CH_L)], nbuf.at[b],
                              sems[b]).wait()

        @plsc.parallel_loop(0, CH_L, step=16, unroll=UNROLL)
        def _grp(j):
            s = ebuf[b, 0, pl.ds(j, 16)]
            d = ebuf[b, 1, pl.ds(j, 16)]
            n = nbuf[b, pl.ds(j, 16)]
            for f in range(FPT):
                hu = plsc.load_gather(hu_v, [cvs[f], s])
                plsc.addupdate_scatter(aggi_v, [cvs[f], d], n * hu)
                hi = plsc.load_gather(hi_v, [cvs[f], d])
                plsc.addupdate_scatter(aggu_v, [cvs[f], s], n * hi)

    def pair(p, c):
        do_chunk(2 * p, 0)
        do_chunk(2 * p + 1, 1)
        return c
    lax.fori_loop(0, NCH_L // 2, pair, 0)

    pltpu.sync_copy(aggu_v, aggu_hbm.at[pl.ds(r0, FPT)])
    pltpu.sync_copy(aggi_v, aggi_hbm.at[pl.ds(r0, FPT)])


def _norm_body(scale, aggu_ref, aggi_ref, embu_ref, embi_ref,
               hu_out, hi_out, embu_out, embi_out):
    for agg_ref, emb_ref, h_out, e_out in (
            (aggu_ref, embu_ref, hu_out, embu_out),
            (aggi_ref, embi_ref, hi_out, embi_out)):
        x = agg_ref[...]
        nrm = jnp.sqrt(jnp.sum(x * x, axis=0, keepdims=True))
        h = x / jnp.maximum(nrm, 1e-12)
        h_out[...] = h
        e_out[...] = emb_ref[...] + h * scale


def _norm_call(scale):
    shp = jax.ShapeDtypeStruct((D, NPAD), jnp.float32)
    return pl.pallas_call(
        functools.partial(_norm_body, scale),
        out_shape=[shp, shp, shp, shp],
    )


@functools.partial(
    pl.kernel,
    out_type=[jax.ShapeDtypeStruct((4096, D), jnp.float32)] * 3,
    mesh=_mesh(),
    compiler_params=_sc_params,
    scratch_types=[
        pltpu.VMEM((GPT,), jnp.int32),
        pltpu.VMEM((GPT, D), jnp.float32),
        pltpu.SemaphoreType.DMA,
    ],
)
def _lookup(embu_hbm, embi_hbm, users_hbm, pos_hbm, neg_hbm,
            ug_hbm, pg_hbm, ng_hbm, idx_v, rows_v, sem):
    wid = _wid()
    base = wid * GPT
    for tab, idx_hbm, out_hbm in ((embu_hbm, users_hbm, ug_hbm),
                                  (embi_hbm, pos_hbm, pg_hbm),
                                  (embi_hbm, neg_hbm, ng_hbm)):
        pltpu.sync_copy(idx_hbm.at[pl.ds(base, GPT)], idx_v)
        pltpu.async_copy(tab.at[idx_v], rows_v, sem).wait()
        pltpu.sync_copy(rows_v, out_hbm.at[pl.ds(base, GPT)])


def kernel(user_feat, item_feat, edge_src, edge_dst, users, pos_items,
           neg_items):
    pad = ((0, 0), (0, NPAD - NU))
    hu_t = jnp.pad(user_feat.T, pad)
    hi_t = jnp.pad(item_feat.T, pad)

    pu, pi = _bincount(edge_src, edge_dst)
    du, di = _degreduce(pu, pi)
    norm_e = _edgenorm(edge_src, edge_dst, du, di)

    embu_t, embi_t = hu_t, hi_t
    for k in range(NLAYERS):
        aggu_t, aggi_t = _layer(hu_t, hi_t, edge_src, edge_dst, norm_e)
        hu_t, hi_t, embu_t, embi_t = _norm_call(1.0 / (k + 1))(
            aggu_t, aggi_t, embu_t, embi_t)

    embu = embu_t[:, :NU].T
    embi = embi_t[:, :NI].T
    return tuple(_lookup(embu, embi, users, pos_items, neg_items))
